# Initial kernel scaffold; baseline (speedup 1.0000x reference)
import functools

import jax
import jax.numpy as jnp
from jax import lax
from jax.experimental import pallas as pl
from jax.experimental.pallas import tpu as pltpu
from jax.experimental.pallas import tpu_sc as plsc

N_NODES = 10000
N_EDGES = 320000
D = 128

NC = 2
NS = 16
NW = NC * NS
EDGES_PER_TILE = N_EDGES // NW
CHUNK = 80
N_CHUNKS = EDGES_PER_TILE // CHUNK
ROWS_PER_SUB = N_NODES // NS


def _sc_gather_scatter(feature, src, dst):
    mesh = plsc.VectorSubcoreMesh(core_axis_name="c", subcore_axis_name="s")

    @functools.partial(
        pl.kernel,
        mesh=mesh,
        out_type=jax.ShapeDtypeStruct((NC, N_NODES, D), jnp.float32),
        scratch_types=[
            pltpu.VMEM((CHUNK,), jnp.int32),
            pltpu.VMEM((CHUNK,), jnp.int32),
            pltpu.VMEM((CHUNK, D), jnp.float32),
            pltpu.VMEM_SHARED((N_NODES, D), jnp.float32),
            pltpu.SemaphoreType.DMA,
        ],
    )
    def k(feature_hbm, src_hbm, dst_hbm, out_hbm, src_v, dst_v, rows_v, acc, sem):
        c = lax.axis_index("c")
        s = lax.axis_index("s")
        wid = s * NC + c

        def zero_row(i, carry):
            for j in range(D // 16):
                rows_v[i, pl.ds(j * 16, 16)] = jnp.zeros((16,), jnp.float32)
            return carry

        lax.fori_loop(0, CHUNK, zero_row, 0)
        rbase = s * ROWS_PER_SUB
        for t in range(ROWS_PER_SUB // CHUNK):
            pltpu.sync_copy(rows_v, acc.at[pl.ds(rbase + t * CHUNK, CHUNK)])
        rem = ROWS_PER_SUB % CHUNK
        if rem:
            pltpu.sync_copy(
                rows_v.at[pl.ds(0, rem)],
                acc.at[pl.ds(rbase + (ROWS_PER_SUB // CHUNK) * CHUNK, rem)],
            )
        plsc.subcore_barrier()

        ebase = wid * EDGES_PER_TILE

        def body(i, carry):
            base = ebase + i * CHUNK
            pltpu.sync_copy(src_hbm.at[pl.ds(base, CHUNK)], src_v)
            pltpu.sync_copy(dst_hbm.at[pl.ds(base, CHUNK)], dst_v)
            pltpu.async_copy(feature_hbm.at[src_v], rows_v, sem).wait()
            pltpu.sync_copy(rows_v, acc.at[dst_v], add=True)
            return carry

        lax.fori_loop(0, N_CHUNKS, body, 0)
        plsc.subcore_barrier()

        pltpu.sync_copy(
            acc.at[pl.ds(rbase, ROWS_PER_SUB)],
            out_hbm.at[c, pl.ds(rbase, ROWS_PER_SUB)],
        )

    return k(feature, src, dst)


BLK = 1000


def _tc_body(hp_ref, g_ref, be_ref, w_ref, b_ref, o_ref):
    h = hp_ref[0] + hp_ref[1]
    mean = jnp.mean(h, axis=-1, keepdims=True)
    var = jnp.mean((h - mean) ** 2, axis=-1, keepdims=True)
    hn = (h - mean) * lax.rsqrt(var + 1e-5)
    hn = hn * g_ref[...] + be_ref[...]
    o_ref[...] = (
        lax.dot_general(hn, w_ref[...], (((1,), (1,)), ((), ())),
                        preferred_element_type=jnp.float32)
        + b_ref[...]
    )


def _tc_finish(hpart, ln_gamma, ln_beta, W, b):
    grid = N_NODES // BLK
    return pl.pallas_call(
        _tc_body,
        grid=(grid,),
        in_specs=[
            pl.BlockSpec((NC, BLK, D), lambda i: (0, i, 0)),
            pl.BlockSpec((1, D), lambda i: (0, 0)),
            pl.BlockSpec((1, D), lambda i: (0, 0)),
            pl.BlockSpec((D, D), lambda i: (0, 0)),
            pl.BlockSpec((1, D), lambda i: (0, 0)),
        ],
        out_specs=pl.BlockSpec((BLK, D), lambda i: (i, 0)),
        out_shape=jax.ShapeDtypeStruct((N_NODES, D), jnp.float32),
    )(hpart, ln_gamma.reshape(1, D), ln_beta.reshape(1, D), W, b.reshape(1, D))


def kernel(feature, edge_index, ln_gamma, ln_beta, W, b):
    ei = edge_index.astype(jnp.int32)
    hpart = _sc_gather_scatter(feature, ei[0], ei[1])
    return _tc_finish(hpart, ln_gamma, ln_beta, W, b)


# speedup vs baseline: 5.5943x; 5.5943x over previous
import functools

import jax
import jax.numpy as jnp
from jax import lax
from jax.experimental import pallas as pl
from jax.experimental.pallas import tpu as pltpu
from jax.experimental.pallas import tpu_sc as plsc

N_NODES = 10000
N_EDGES = 320000
D = 128

NC = 2
NS = 16
NW = NC * NS
EDGES_PER_TILE = N_EDGES // NW
CHUNK = 80
N_CHUNKS = EDGES_PER_TILE // CHUNK
N_PAD = 10240
ROWS_PER_SUB = N_PAD // NS


def _sc_gather_scatter(feature, src, dst):
    mesh = plsc.VectorSubcoreMesh(core_axis_name="c", subcore_axis_name="s")

    @functools.partial(
        pl.kernel,
        mesh=mesh,
        out_type=jax.ShapeDtypeStruct((NC, N_PAD, D), jnp.float32),
        scratch_types=[
            pltpu.VMEM((CHUNK,), jnp.int32),
            pltpu.VMEM((CHUNK,), jnp.int32),
            pltpu.VMEM((CHUNK, D), jnp.float32),
            pltpu.VMEM_SHARED((N_PAD, D), jnp.float32),
            pltpu.SemaphoreType.DMA,
        ],
    )
    def k(feature_hbm, src_hbm, dst_hbm, out_hbm, src_v, dst_v, rows_v, acc, sem):
        c = lax.axis_index("c")
        s = lax.axis_index("s")
        wid = s * NC + c

        def zero_row(i, carry):
            for j in range(D // 16):
                rows_v[i, pl.ds(j * 16, 16)] = jnp.zeros((16,), jnp.float32)
            return carry

        lax.fori_loop(0, CHUNK, zero_row, 0)
        rbase = s * ROWS_PER_SUB
        for t in range(ROWS_PER_SUB // CHUNK):
            pltpu.sync_copy(rows_v, acc.at[pl.ds(rbase + t * CHUNK, CHUNK)])
        plsc.subcore_barrier()

        ebase = wid * EDGES_PER_TILE

        def body(i, carry):
            base = ebase + i * CHUNK
            pltpu.sync_copy(src_hbm.at[pl.ds(base, CHUNK)], src_v)
            pltpu.sync_copy(dst_hbm.at[pl.ds(base, CHUNK)], dst_v)
            pltpu.async_copy(feature_hbm.at[src_v], rows_v, sem).wait()
            pltpu.sync_copy(rows_v, acc.at[dst_v], add=True)
            return carry

        lax.fori_loop(0, N_CHUNKS, body, 0)
        plsc.subcore_barrier()

        pltpu.sync_copy(
            acc.at[pl.ds(rbase, ROWS_PER_SUB)],
            out_hbm.at[c, pl.ds(rbase, ROWS_PER_SUB)],
        )

    return k(feature, src, dst)


BLK = 1000


def _tc_body(hp_ref, g_ref, be_ref, w_ref, b_ref, o_ref):
    h = hp_ref[0] + hp_ref[1]
    mean = jnp.mean(h, axis=-1, keepdims=True)
    var = jnp.mean((h - mean) ** 2, axis=-1, keepdims=True)
    hn = (h - mean) * lax.rsqrt(var + 1e-5)
    hn = hn * g_ref[...] + be_ref[...]
    o_ref[...] = (
        lax.dot_general(hn, w_ref[...], (((1,), (1,)), ((), ())),
                        preferred_element_type=jnp.float32)
        + b_ref[...]
    )


def _tc_finish(hpart, ln_gamma, ln_beta, W, b):
    grid = N_NODES // BLK
    return pl.pallas_call(
        _tc_body,
        grid=(grid,),
        in_specs=[
            pl.BlockSpec((NC, BLK, D), lambda i: (0, i, 0)),
            pl.BlockSpec((1, D), lambda i: (0, 0)),
            pl.BlockSpec((1, D), lambda i: (0, 0)),
            pl.BlockSpec((D, D), lambda i: (0, 0)),
            pl.BlockSpec((1, D), lambda i: (0, 0)),
        ],
        out_specs=pl.BlockSpec((BLK, D), lambda i: (i, 0)),
        out_shape=jax.ShapeDtypeStruct((N_NODES, D), jnp.float32),
    )(hpart, ln_gamma.reshape(1, D), ln_beta.reshape(1, D), W, b.reshape(1, D))


def kernel(feature, edge_index, ln_gamma, ln_beta, W, b):
    ei = edge_index.astype(jnp.int32)
    hpart = _sc_gather_scatter(feature, ei[0], ei[1])
    return _tc_finish(hpart, ln_gamma, ln_beta, W, b)


# speedup vs baseline: 9.9888x; 1.7855x over previous
import functools

import jax
import jax.numpy as jnp
from jax import lax
from jax.experimental import pallas as pl
from jax.experimental.pallas import tpu as pltpu
from jax.experimental.pallas import tpu_sc as plsc

N_NODES = 10000
N_EDGES = 320000
D = 128

NC = 2
NS = 16
NW = NC * NS
EDGES_PER_TILE = N_EDGES // NW
CHUNK = 80
N_CHUNKS = EDGES_PER_TILE // CHUNK
N_PAD = 10240
ROWS_PER_SUB = N_PAD // NS


def _sc_gather_scatter(feature, src3, dst3):
    mesh = plsc.VectorSubcoreMesh(core_axis_name="c", subcore_axis_name="s")

    @functools.partial(
        pl.kernel,
        mesh=mesh,
        out_type=jax.ShapeDtypeStruct((NC, N_PAD, D), jnp.float32),
        scratch_types=[
            pltpu.VMEM((CHUNK,), jnp.int32),
            pltpu.VMEM((CHUNK,), jnp.int32),
            pltpu.VMEM((CHUNK,), jnp.int32),
            pltpu.VMEM((CHUNK,), jnp.int32),
            pltpu.VMEM((CHUNK, D), jnp.float32),
            pltpu.VMEM((CHUNK, D), jnp.float32),
            pltpu.VMEM_SHARED((N_PAD, D), jnp.float32),
            pltpu.SemaphoreType.DMA,
            pltpu.SemaphoreType.DMA,
            pltpu.SemaphoreType.DMA,
            pltpu.SemaphoreType.DMA,
            pltpu.SemaphoreType.DMA,
            pltpu.SemaphoreType.DMA,
        ],
    )
    def k(feature_hbm, src_hbm, dst_hbm, out_hbm,
          src0, src1, dst0, dst1, rows0, rows1, acc,
          sg0, sg1, si0, si1, sd0, sd1):
        c = lax.axis_index("c")
        s = lax.axis_index("s")
        wid = s * NC + c
        rbase = s * ROWS_PER_SUB
        ebase = wid * EDGES_PER_TILE
        src_i = (src0, src1)
        dst_i = (dst0, dst1)
        rows = (rows0, rows1)
        sg = (sg0, sg1)
        si = (si0, si1)
        sd = (sd0, sd1)

        def zero_row(i, carry):
            for j in range(D // 16):
                rows0[i, pl.ds(j * 16, 16)] = jnp.zeros((16,), jnp.float32)
            return carry

        lax.fori_loop(0, CHUNK, zero_row, 0)
        for t in range(ROWS_PER_SUB // CHUNK):
            pltpu.sync_copy(rows0, acc.at[pl.ds(rbase + t * CHUNK, CHUNK)])
        plsc.subcore_barrier()

        def step(n, b):
            gcp = pltpu.async_copy(
                feature_hbm.at[src_i[b]], rows[b], sg[b])
            pltpu.sync_copy(
                rows[1 - b], acc.at[dst_i[1 - b]], add=True)
            nb = jnp.minimum(n + 1, N_CHUNKS - 1) * CHUNK + ebase
            ic0 = pltpu.async_copy(
                src_hbm.at[pl.ds(nb, CHUNK)], src_i[1 - b], si[1 - b])
            ic1 = pltpu.async_copy(
                dst_hbm.at[pl.ds(nb, CHUNK)], dst_i[1 - b], sd[1 - b])
            gcp.wait()
            ic0.wait()
            ic1.wait()

        pltpu.sync_copy(src_hbm.at[pl.ds(ebase, CHUNK)], src0)
        pltpu.sync_copy(dst_hbm.at[pl.ds(ebase, CHUNK)], dst0)
        gcp = pltpu.async_copy(feature_hbm.at[src0], rows0, sg0)
        ic0 = pltpu.async_copy(
            src_hbm.at[pl.ds(ebase + CHUNK, CHUNK)], src1, si1)
        ic1 = pltpu.async_copy(
            dst_hbm.at[pl.ds(ebase + CHUNK, CHUNK)], dst1, sd1)
        gcp.wait()
        ic0.wait()
        ic1.wait()

        def body(g, carry):
            step(2 * g + 1, 1)
            step(2 * g + 2, 0)
            return carry

        lax.fori_loop(0, (N_CHUNKS - 1) // 2, body, 0)
        pltpu.sync_copy(rows0, acc.at[dst0], add=True)
        plsc.subcore_barrier()

        pltpu.sync_copy(
            acc.at[pl.ds(rbase, ROWS_PER_SUB)],
            out_hbm.at[c, pl.ds(rbase, ROWS_PER_SUB)],
        )

    return k(feature, src3, dst3)


BLK = 1000


def _tc_body(hp_ref, g_ref, be_ref, w_ref, b_ref, o_ref):
    h = hp_ref[0] + hp_ref[1]
    mean = jnp.mean(h, axis=-1, keepdims=True)
    var = jnp.mean((h - mean) ** 2, axis=-1, keepdims=True)
    hn = (h - mean) * lax.rsqrt(var + 1e-5)
    hn = hn * g_ref[...] + be_ref[...]
    o_ref[...] = (
        lax.dot_general(hn, w_ref[...], (((1,), (1,)), ((), ())),
                        preferred_element_type=jnp.float32)
        + b_ref[...]
    )


def _tc_finish(hpart, ln_gamma, ln_beta, W, b):
    grid = N_NODES // BLK
    return pl.pallas_call(
        _tc_body,
        grid=(grid,),
        in_specs=[
            pl.BlockSpec((NC, BLK, D), lambda i: (0, i, 0)),
            pl.BlockSpec((1, D), lambda i: (0, 0)),
            pl.BlockSpec((1, D), lambda i: (0, 0)),
            pl.BlockSpec((D, D), lambda i: (0, 0)),
            pl.BlockSpec((1, D), lambda i: (0, 0)),
        ],
        out_specs=pl.BlockSpec((BLK, D), lambda i: (i, 0)),
        out_shape=jax.ShapeDtypeStruct((N_NODES, D), jnp.float32),
    )(hpart, ln_gamma.reshape(1, D), ln_beta.reshape(1, D), W, b.reshape(1, D))


def kernel(feature, edge_index, ln_gamma, ln_beta, W, b):
    ei = edge_index.astype(jnp.int32)
    hpart = _sc_gather_scatter(feature, ei[0], ei[1])
    return _tc_finish(hpart, ln_gamma, ln_beta, W, b)
